# Initial kernel scaffold; baseline (speedup 1.0000x reference)
import jax
import jax.numpy as jnp
from jax.experimental import pallas as pl


def kernel(input_ids, codebook_idxs, embed_audio_tokens_weight):
    raise NotImplementedError("write your pallas kernel here")


# speedup vs baseline: 1.5494x; 1.5494x over previous
import functools

import jax
import jax.numpy as jnp
from jax import lax
from jax.experimental import pallas as pl
from jax.experimental.pallas import tpu as pltpu
from jax.experimental.pallas import tpu_sc as plsc

_VOCAB = 2051
_LANES = 16

_NC = 2
_NS = 16
_NW = _NC * _NS

_CHUNK = 32


@functools.lru_cache(maxsize=None)
def _build(n_tokens: int, vocab_rows: int, d: int):
    per_w = n_tokens // _NW
    n_chunks = per_w // _CHUNK
    mesh = plsc.VectorSubcoreMesh(core_axis_name="c", subcore_axis_name="s")

    @functools.partial(
        pl.kernel,
        out_type=jax.ShapeDtypeStruct((n_tokens, d), jnp.float32),
        mesh=mesh,
        scratch_types=[
            pltpu.VMEM((per_w,), jnp.int32),
            pltpu.VMEM((per_w,), jnp.int32),
            pltpu.VMEM((n_chunks, _CHUNK), jnp.int32),
            pltpu.VMEM((_CHUNK, d), jnp.float32),
            pltpu.SemaphoreType.DMA,
        ],
    )
    def gather_kernel(ids_hbm, cbs_hbm, table_hbm, out_hbm,
                      ids_v, cbs_v, idx_v, rows_v, sem):
        wid = lax.axis_index("s") * _NC + lax.axis_index("c")
        base = wid * per_w

        pltpu.sync_copy(ids_hbm.at[pl.ds(base, per_w)], ids_v)
        pltpu.sync_copy(cbs_hbm.at[pl.ds(base, per_w)], cbs_v)

        for i in range(per_w // _LANES):
            p = i * _LANES
            flat = ids_v[pl.ds(p, _LANES)] + cbs_v[pl.ds(p, _LANES)] * _VOCAB
            idx_v[p // _CHUNK, pl.ds(p % _CHUNK, _LANES)] = flat

        for j in range(n_chunks):
            pltpu.async_copy(table_hbm.at[idx_v.at[j]], rows_v, sem).wait()
            pltpu.sync_copy(rows_v, out_hbm.at[pl.ds(base + j * _CHUNK, _CHUNK)])

    return gather_kernel


def kernel(input_ids, codebook_idxs, embed_audio_tokens_weight):
    b, s = input_ids.shape
    vocab_rows, d = embed_audio_tokens_weight.shape
    flat_ids = input_ids.reshape(-1).astype(jnp.int32)
    flat_cbs = codebook_idxs.reshape(-1).astype(jnp.int32)
    out = _build(b * s, vocab_rows, d)(flat_ids, flat_cbs, embed_audio_tokens_weight)
    return out.reshape(b, s, d)


# speedup vs baseline: 1.6599x; 1.0713x over previous
import functools

import jax
import jax.numpy as jnp
from jax import lax
from jax.experimental import pallas as pl
from jax.experimental.pallas import tpu as pltpu
from jax.experimental.pallas import tpu_sc as plsc

_VOCAB = 2051
_LANES = 16

_NC = 2
_NS = 16
_NW = _NC * _NS

_CHUNK = 16
_NBUF = 3


@functools.lru_cache(maxsize=None)
def _build(n_tokens: int, vocab_rows: int, d: int):
    per_w = n_tokens // _NW
    n_chunks = per_w // _CHUNK
    mesh = plsc.VectorSubcoreMesh(core_axis_name="c", subcore_axis_name="s")

    @functools.partial(
        pl.kernel,
        out_type=jax.ShapeDtypeStruct((n_tokens, d), jnp.float32),
        mesh=mesh,
        scratch_types=[
            pltpu.VMEM((per_w,), jnp.int32),
            pltpu.VMEM((per_w,), jnp.int32),
            pltpu.VMEM((n_chunks, _CHUNK), jnp.int32),
        ]
        + [pltpu.VMEM((_CHUNK, d), jnp.float32) for _ in range(_NBUF)]
        + [pltpu.SemaphoreType.DMA for _ in range(2 * _NBUF)],
    )
    def gather_kernel(ids_hbm, cbs_hbm, table_hbm, out_hbm,
                      ids_v, cbs_v, idx_v, *bufs_and_sems):
        bufs = bufs_and_sems[:_NBUF]
        gsems = bufs_and_sems[_NBUF:2 * _NBUF]
        wsems = bufs_and_sems[2 * _NBUF:]
        wid = lax.axis_index("s") * _NC + lax.axis_index("c")
        base = wid * per_w

        pltpu.sync_copy(ids_hbm.at[pl.ds(base, per_w)], ids_v)
        pltpu.sync_copy(cbs_hbm.at[pl.ds(base, per_w)], cbs_v)

        for i in range(per_w // _LANES):
            p = i * _LANES
            flat = ids_v[pl.ds(p, _LANES)] + cbs_v[pl.ds(p, _LANES)] * _VOCAB
            idx_v[p // _CHUNK, pl.ds(p % _CHUNK, _LANES)] = flat

        def start_gather(j):
            return pltpu.async_copy(
                table_hbm.at[idx_v.at[j]], bufs[j % _NBUF], gsems[j % _NBUF])

        def start_write(j):
            return pltpu.async_copy(
                bufs[j % _NBUF],
                out_hbm.at[pl.ds(base + j * _CHUNK, _CHUNK)],
                wsems[j % _NBUF])

        g = [None] * n_chunks
        w = [None] * n_chunks
        for j in range(min(_NBUF - 1, n_chunks)):
            g[j] = start_gather(j)
        for j in range(n_chunks):
            nxt = j + _NBUF - 1
            if nxt < n_chunks:
                if nxt - _NBUF >= 0:
                    w[nxt - _NBUF].wait()
                g[nxt] = start_gather(nxt)
            g[j].wait()
            w[j] = start_write(j)
        for j in range(max(0, n_chunks - _NBUF), n_chunks):
            w[j].wait()

    return gather_kernel


def kernel(input_ids, codebook_idxs, embed_audio_tokens_weight):
    b, s = input_ids.shape
    vocab_rows, d = embed_audio_tokens_weight.shape
    flat_ids = input_ids.reshape(-1).astype(jnp.int32)
    flat_cbs = codebook_idxs.reshape(-1).astype(jnp.int32)
    out = _build(b * s, vocab_rows, d)(flat_ids, flat_cbs, embed_audio_tokens_weight)
    return out.reshape(b, s, d)
